# Initial kernel scaffold; baseline (speedup 1.0000x reference)
import jax
import jax.numpy as jnp
from jax.experimental import pallas as pl


def kernel(x_c, x_m, x_d, e0, e1, e2, e3, e4, e5, Wl1, Wr1, b1, Wl, Wr, b):
    raise NotImplementedError("write your pallas kernel here")


# speedup vs baseline: 4.5357x; 4.5357x over previous
import functools

import jax
import jax.numpy as jnp
from jax import lax
from jax.experimental import pallas as pl
from jax.experimental.pallas import tpu as pltpu
from jax.experimental.pallas import tpu_sc as plsc

N = 10000
E = 160000
D_IN = 128
D_H = 64

NC = 2
NS = 16
NP = 10112
SLICE = NP // NS
CH = 128
EPT = (E + NS - 1) // NS
NCHK = -(-(-(-EPT // CH)) // 8) * 8
EPT_PAD = NCHK * CH
E_PAD = EPT_PAD * NS

BN = 1264
GRID = NP // BN

SRC_T = (0, 1, 0, 1, 2, 2)
PAIRS = ((3, 5), (0, 4), (1, 2))

_mesh = plsc.VectorSubcoreMesh(
    core_axis_name="c", subcore_axis_name="s", num_cores=NC, num_subcores=NS)
_sc_params = pltpu.CompilerParams(use_tc_tiling_on_sc=False)


@functools.partial(
    pl.kernel,
    out_type=jax.ShapeDtypeStruct((6, NP, D_H), jnp.float32),
    mesh=_mesh,
    compiler_params=_sc_params,
    scratch_types=[
        pltpu.VMEM((NCHK, CH), jnp.int32),
        pltpu.VMEM((NCHK, CH), jnp.int32),
        pltpu.VMEM((CH, D_H), jnp.float32),
        pltpu.VMEM_SHARED((NP, D_H), jnp.float32),
        pltpu.SemaphoreType.DMA,
    ],
)
def _sc_aggregate(y_hbm, src_hbm, dst_hbm, z_hbm, out_hbm,
                  sidx, didx, rows, acc, sem):
    c = lax.axis_index("c")
    s = lax.axis_index("s")
    for rl in range(3):
        r = c * 3 + rl
        pltpu.sync_copy(z_hbm, acc.at[pl.ds(s * SLICE, SLICE)])
        pltpu.sync_copy(src_hbm.at[r, pl.ds(s * NCHK, NCHK)], sidx)
        pltpu.sync_copy(dst_hbm.at[r, pl.ds(s * NCHK, NCHK)], didx)
        plsc.subcore_barrier()

        def chunk(k, carry):
            pltpu.async_copy(y_hbm.at[sidx.at[k]], rows, sem).wait()
            pltpu.sync_copy(rows, acc.at[didx.at[k]], add=True)
            return carry

        lax.fori_loop(0, NCHK, chunk, 0)
        plsc.subcore_barrier()
        pltpu.sync_copy(
            acc.at[pl.ds(s * SLICE, SLICE)],
            out_hbm.at[r, pl.ds(s * SLICE, SLICE)])


@functools.partial(
    pl.kernel,
    out_type=jax.ShapeDtypeStruct((6, NP, 16), jnp.float32),
    mesh=_mesh,
    compiler_params=_sc_params,
    scratch_types=[
        pltpu.VMEM((NCHK, CH), jnp.int32),
        pltpu.VMEM((CH, 16), jnp.float32),
        pltpu.VMEM_SHARED((NP, 16), jnp.float32),
    ],
)
def _sc_count(dst_hbm, ones_hbm, z_hbm, out_hbm, didx, ones_v, acc):
    c = lax.axis_index("c")
    s = lax.axis_index("s")
    pltpu.sync_copy(ones_hbm, ones_v)
    for rl in range(3):
        r = c * 3 + rl
        pltpu.sync_copy(z_hbm, acc.at[pl.ds(s * SLICE, SLICE)])
        pltpu.sync_copy(dst_hbm.at[r, pl.ds(s * NCHK, NCHK)], didx)
        plsc.subcore_barrier()

        def chunk(k, carry):
            pltpu.sync_copy(ones_v, acc.at[didx.at[k]], add=True)
            return carry

        lax.fori_loop(0, NCHK, chunk, 0)
        plsc.subcore_barrier()
        pltpu.sync_copy(
            acc.at[pl.ds(s * SLICE, SLICE)],
            out_hbm.at[r, pl.ds(s * SLICE, SLICE)])


def _matT(x, w):
    return lax.dot_general(x, w, (((1,), (1,)), ((), ())),
                           preferred_element_type=jnp.float32)


def _tc_first_body(xc_ref, xm_ref, xd_ref, wl_ref, wr_ref, b_ref,
                   y_ref, r_ref):
    xs = [xc_ref[...], xm_ref[...], xd_ref[...]]
    bfull = b_ref[...]
    for r in range(6):
        y_ref[r] = _matT(xs[SRC_T[r]], wl_ref[r])
    for t, (a, b2) in enumerate(PAIRS):
        wrm = 0.5 * (wr_ref[a] + wr_ref[b2])
        bm = 0.5 * (bfull[a:a + 1, :] + bfull[b2:b2 + 1, :])
        r_ref[t] = _matT(xs[t], wrm) + bm


def _combine(s_ref, cnt_ref, root_ref, t, relu):
    a, b2 = PAIRS[t]
    inva = 1.0 / jnp.maximum(cnt_ref[a, :, 0:1], 1.0)
    invb = 1.0 / jnp.maximum(cnt_ref[b2, :, 0:1], 1.0)
    h = 0.5 * (s_ref[a] * inva + s_ref[b2] * invb) + root_ref[t]
    return jnp.maximum(h, 0.0) if relu else h


def _tc_mid_body(s_ref, cnt_ref, root_ref, wl_ref, wr_ref, b_ref,
                 y_ref, r_ref):
    hs = [_combine(s_ref, cnt_ref, root_ref, t, True) for t in range(3)]
    bfull = b_ref[...]
    for r in range(6):
        y_ref[r] = _matT(hs[SRC_T[r]], wl_ref[r])
    for t, (a, b2) in enumerate(PAIRS):
        wrm = 0.5 * (wr_ref[a] + wr_ref[b2])
        bm = 0.5 * (bfull[a:a + 1, :] + bfull[b2:b2 + 1, :])
        r_ref[t] = _matT(hs[t], wrm) + bm


def _tc_final_body(s_ref, cnt_ref, root_ref, out_ref):
    for t in range(3):
        h = _combine(s_ref, cnt_ref, root_ref, t, False)
        nrm = jnp.sqrt(jnp.sum(h * h, axis=1, keepdims=True))
        out_ref[t] = h / jnp.maximum(nrm, 1e-12)


def _full(shape):
    nd = len(shape)
    return pl.BlockSpec(shape, lambda i, _n=nd: (0,) * _n)


def _tc_first(xc, xm, xd, wl, wr, b):
    return pl.pallas_call(
        _tc_first_body,
        grid=(GRID,),
        in_specs=[
            pl.BlockSpec((BN, D_IN), lambda i: (i, 0)),
            pl.BlockSpec((BN, D_IN), lambda i: (i, 0)),
            pl.BlockSpec((BN, D_IN), lambda i: (i, 0)),
            _full(wl.shape), _full(wr.shape), _full(b.shape),
        ],
        out_specs=[
            pl.BlockSpec((6, BN, D_H), lambda i: (0, i, 0)),
            pl.BlockSpec((3, BN, D_H), lambda i: (0, i, 0)),
        ],
        out_shape=[
            jax.ShapeDtypeStruct((6, NP, D_H), jnp.float32),
            jax.ShapeDtypeStruct((3, NP, D_H), jnp.float32),
        ],
    )(xc, xm, xd, wl, wr, b)


def _tc_mid(s, cnt, root, wl, wr, b):
    return pl.pallas_call(
        _tc_mid_body,
        grid=(GRID,),
        in_specs=[
            pl.BlockSpec((6, BN, D_H), lambda i: (0, i, 0)),
            pl.BlockSpec((6, BN, 16), lambda i: (0, i, 0)),
            pl.BlockSpec((3, BN, D_H), lambda i: (0, i, 0)),
            _full(wl.shape), _full(wr.shape), _full(b.shape),
        ],
        out_specs=[
            pl.BlockSpec((6, BN, D_H), lambda i: (0, i, 0)),
            pl.BlockSpec((3, BN, D_H), lambda i: (0, i, 0)),
        ],
        out_shape=[
            jax.ShapeDtypeStruct((6, NP, D_H), jnp.float32),
            jax.ShapeDtypeStruct((3, NP, D_H), jnp.float32),
        ],
    )(s, cnt, root, wl, wr, b)


def _tc_final(s, cnt, root):
    return pl.pallas_call(
        _tc_final_body,
        grid=(GRID,),
        in_specs=[
            pl.BlockSpec((6, BN, D_H), lambda i: (0, i, 0)),
            pl.BlockSpec((6, BN, 16), lambda i: (0, i, 0)),
            pl.BlockSpec((3, BN, D_H), lambda i: (0, i, 0)),
        ],
        out_specs=pl.BlockSpec((3, BN, D_H), lambda i: (0, i, 0)),
        out_shape=jax.ShapeDtypeStruct((3, NP, D_H), jnp.float32),
    )(s, cnt, root)


def kernel(x_c, x_m, x_d, e0, e1, e2, e3, e4, e5, Wl1, Wr1, b1, Wl, Wr, b):
    srcs, dsts = [], []
    for r, e in enumerate((e0, e1, e2, e3, e4, e5)):
        src = e[0].astype(jnp.int32) + r * NP
        dst = e[1].astype(jnp.int32)
        pad_src = jnp.full((E_PAD - E,), r * NP, jnp.int32)
        pad_dst = jnp.full((E_PAD - E,), N, jnp.int32)
        srcs.append(jnp.concatenate([src, pad_src]))
        dsts.append(jnp.concatenate([dst, pad_dst]))
    src_all = jnp.stack(srcs).reshape(6, NS * NCHK, CH)
    dst_all = jnp.stack(dsts).reshape(6, NS * NCHK, CH)

    zeros64 = jnp.zeros((SLICE, D_H), jnp.float32)
    zeros16 = jnp.zeros((SLICE, 16), jnp.float32)
    ones16 = jnp.ones((CH, 16), jnp.float32)

    cnt = _sc_count(dst_all, ones16, zeros16)

    pad_rows = ((0, NP - N), (0, 0))
    xc = jnp.pad(x_c, pad_rows)
    xm = jnp.pad(x_m, pad_rows)
    xd = jnp.pad(x_d, pad_rows)

    y, root = _tc_first(xc, xm, xd, Wl1, Wr1, b1)
    for i in range(3):
        s = _sc_aggregate(y.reshape(6 * NP, D_H), src_all, dst_all, zeros64)
        y, root = _tc_mid(s, cnt, root, Wl[i], Wr[i], b[i])
    s = _sc_aggregate(y.reshape(6 * NP, D_H), src_all, dst_all, zeros64)
    out = _tc_final(s, cnt, root)
    return out[:, :N, :]


# speedup vs baseline: 5.9221x; 1.3057x over previous
import functools

import jax
import jax.numpy as jnp
from jax import lax
from jax.experimental import pallas as pl
from jax.experimental.pallas import tpu as pltpu
from jax.experimental.pallas import tpu_sc as plsc

N = 10000
E = 160000
D_IN = 128
D_H = 64

NC = 2
NS = 16
NP = 10112
SLICE = NP // NS
CH = 128
EPT = (E + NS - 1) // NS
NCHK = -(-(-(-EPT // CH)) // 8) * 8
EPT_PAD = NCHK * CH
E_PAD = EPT_PAD * NS

BN = 1264
GRID = NP // BN

SRC_T = (0, 1, 0, 1, 2, 2)
PAIRS = ((3, 5), (0, 4), (1, 2))

_mesh = plsc.VectorSubcoreMesh(
    core_axis_name="c", subcore_axis_name="s", num_cores=NC, num_subcores=NS)
_sc_params = pltpu.CompilerParams(use_tc_tiling_on_sc=False)


@functools.partial(
    pl.kernel,
    out_type=jax.ShapeDtypeStruct((6, NP, D_H), jnp.float32),
    mesh=_mesh,
    compiler_params=_sc_params,
    scratch_types=[
        pltpu.VMEM((NCHK, CH), jnp.int32),
        pltpu.VMEM((NCHK, CH), jnp.int32),
        [pltpu.VMEM((CH, D_H), jnp.float32) for _ in range(8)],
        pltpu.VMEM_SHARED((NP, D_H), jnp.float32),
        [pltpu.SemaphoreType.DMA for _ in range(4)],
    ],
)
def _sc_aggregate(y_hbm, src_hbm, dst_hbm, z_hbm, out_hbm,
                  sidx, didx, rows, acc, sems):
    c = lax.axis_index("c")
    s = lax.axis_index("s")
    gsem_a, ssem_a, gsem_b, ssem_b = sems
    bufs_a, bufs_b = rows[:4], rows[4:]

    def start_gathers(k0, bufs, sem):
        for i in range(4):
            pltpu.async_copy(y_hbm.at[sidx.at[k0 + i]], bufs[i], sem)

    def drain_gathers(k0, bufs, sem):
        for i in range(4):
            pltpu.make_async_copy(y_hbm.at[sidx.at[k0 + i]], bufs[i],
                                  sem).wait()

    def run_scatters(k0, bufs, sem):
        descs = [pltpu.async_copy(bufs[i], acc.at[didx.at[k0 + i]], sem,
                                  add=True) for i in range(4)]
        for d in descs:
            d.wait()

    for rl in range(3):
        r = c * 3 + rl
        pltpu.sync_copy(z_hbm, acc.at[pl.ds(s * SLICE, SLICE)])
        pltpu.sync_copy(src_hbm.at[r, pl.ds(s * NCHK, NCHK)], sidx)
        pltpu.sync_copy(dst_hbm.at[r, pl.ds(s * NCHK, NCHK)], didx)
        plsc.subcore_barrier()

        start_gathers(0, bufs_a, gsem_a)

        def step(j, carry):
            k = j * 8
            start_gathers(k + 4, bufs_b, gsem_b)
            drain_gathers(k, bufs_a, gsem_a)
            run_scatters(k, bufs_a, ssem_a)

            @pl.when(j < NCHK // 8 - 1)
            def _():
                start_gathers(k + 8, bufs_a, gsem_a)

            drain_gathers(k + 4, bufs_b, gsem_b)
            run_scatters(k + 4, bufs_b, ssem_b)
            return carry

        lax.fori_loop(0, NCHK // 8, step, 0)
        plsc.subcore_barrier()
        pltpu.sync_copy(
            acc.at[pl.ds(s * SLICE, SLICE)],
            out_hbm.at[r, pl.ds(s * SLICE, SLICE)])


@functools.partial(
    pl.kernel,
    out_type=jax.ShapeDtypeStruct((6, NP, 16), jnp.float32),
    mesh=_mesh,
    compiler_params=_sc_params,
    scratch_types=[
        pltpu.VMEM((NCHK, CH), jnp.int32),
        pltpu.VMEM((CH, 16), jnp.float32),
        pltpu.VMEM_SHARED((NP, 16), jnp.float32),
    ],
)
def _sc_count(dst_hbm, ones_hbm, z_hbm, out_hbm, didx, ones_v, acc):
    c = lax.axis_index("c")
    s = lax.axis_index("s")
    pltpu.sync_copy(ones_hbm, ones_v)
    for rl in range(3):
        r = c * 3 + rl
        pltpu.sync_copy(z_hbm, acc.at[pl.ds(s * SLICE, SLICE)])
        pltpu.sync_copy(dst_hbm.at[r, pl.ds(s * NCHK, NCHK)], didx)
        plsc.subcore_barrier()

        def chunk(k, carry):
            pltpu.sync_copy(ones_v, acc.at[didx.at[k]], add=True)
            return carry

        lax.fori_loop(0, NCHK, chunk, 0)
        plsc.subcore_barrier()
        pltpu.sync_copy(
            acc.at[pl.ds(s * SLICE, SLICE)],
            out_hbm.at[r, pl.ds(s * SLICE, SLICE)])


def _matT(x, w):
    return lax.dot_general(x, w, (((1,), (1,)), ((), ())),
                           preferred_element_type=jnp.float32)


def _tc_first_body(xc_ref, xm_ref, xd_ref, wl_ref, wr_ref, b_ref,
                   y_ref, r_ref):
    xs = [xc_ref[...], xm_ref[...], xd_ref[...]]
    bfull = b_ref[...]
    for r in range(6):
        y_ref[r] = _matT(xs[SRC_T[r]], wl_ref[r])
    for t, (a, b2) in enumerate(PAIRS):
        wrm = 0.5 * (wr_ref[a] + wr_ref[b2])
        bm = 0.5 * (bfull[a:a + 1, :] + bfull[b2:b2 + 1, :])
        r_ref[t] = _matT(xs[t], wrm) + bm


def _combine(s_ref, cnt_ref, root_ref, t, relu):
    a, b2 = PAIRS[t]
    inva = 1.0 / jnp.maximum(cnt_ref[a, :, 0:1], 1.0)
    invb = 1.0 / jnp.maximum(cnt_ref[b2, :, 0:1], 1.0)
    h = 0.5 * (s_ref[a] * inva + s_ref[b2] * invb) + root_ref[t]
    return jnp.maximum(h, 0.0) if relu else h


def _tc_mid_body(s_ref, cnt_ref, root_ref, wl_ref, wr_ref, b_ref,
                 y_ref, r_ref):
    hs = [_combine(s_ref, cnt_ref, root_ref, t, True) for t in range(3)]
    bfull = b_ref[...]
    for r in range(6):
        y_ref[r] = _matT(hs[SRC_T[r]], wl_ref[r])
    for t, (a, b2) in enumerate(PAIRS):
        wrm = 0.5 * (wr_ref[a] + wr_ref[b2])
        bm = 0.5 * (bfull[a:a + 1, :] + bfull[b2:b2 + 1, :])
        r_ref[t] = _matT(hs[t], wrm) + bm


def _tc_final_body(s_ref, cnt_ref, root_ref, out_ref):
    for t in range(3):
        h = _combine(s_ref, cnt_ref, root_ref, t, False)
        nrm = jnp.sqrt(jnp.sum(h * h, axis=1, keepdims=True))
        out_ref[t] = h / jnp.maximum(nrm, 1e-12)


def _full(shape):
    nd = len(shape)
    return pl.BlockSpec(shape, lambda i, _n=nd: (0,) * _n)


def _tc_first(xc, xm, xd, wl, wr, b):
    return pl.pallas_call(
        _tc_first_body,
        grid=(GRID,),
        in_specs=[
            pl.BlockSpec((BN, D_IN), lambda i: (i, 0)),
            pl.BlockSpec((BN, D_IN), lambda i: (i, 0)),
            pl.BlockSpec((BN, D_IN), lambda i: (i, 0)),
            _full(wl.shape), _full(wr.shape), _full(b.shape),
        ],
        out_specs=[
            pl.BlockSpec((6, BN, D_H), lambda i: (0, i, 0)),
            pl.BlockSpec((3, BN, D_H), lambda i: (0, i, 0)),
        ],
        out_shape=[
            jax.ShapeDtypeStruct((6, NP, D_H), jnp.float32),
            jax.ShapeDtypeStruct((3, NP, D_H), jnp.float32),
        ],
    )(xc, xm, xd, wl, wr, b)


def _tc_mid(s, cnt, root, wl, wr, b):
    return pl.pallas_call(
        _tc_mid_body,
        grid=(GRID,),
        in_specs=[
            pl.BlockSpec((6, BN, D_H), lambda i: (0, i, 0)),
            pl.BlockSpec((6, BN, 16), lambda i: (0, i, 0)),
            pl.BlockSpec((3, BN, D_H), lambda i: (0, i, 0)),
            _full(wl.shape), _full(wr.shape), _full(b.shape),
        ],
        out_specs=[
            pl.BlockSpec((6, BN, D_H), lambda i: (0, i, 0)),
            pl.BlockSpec((3, BN, D_H), lambda i: (0, i, 0)),
        ],
        out_shape=[
            jax.ShapeDtypeStruct((6, NP, D_H), jnp.float32),
            jax.ShapeDtypeStruct((3, NP, D_H), jnp.float32),
        ],
    )(s, cnt, root, wl, wr, b)


def _tc_final(s, cnt, root):
    return pl.pallas_call(
        _tc_final_body,
        grid=(GRID,),
        in_specs=[
            pl.BlockSpec((6, BN, D_H), lambda i: (0, i, 0)),
            pl.BlockSpec((6, BN, 16), lambda i: (0, i, 0)),
            pl.BlockSpec((3, BN, D_H), lambda i: (0, i, 0)),
        ],
        out_specs=pl.BlockSpec((3, BN, D_H), lambda i: (0, i, 0)),
        out_shape=jax.ShapeDtypeStruct((3, NP, D_H), jnp.float32),
    )(s, cnt, root)


def kernel(x_c, x_m, x_d, e0, e1, e2, e3, e4, e5, Wl1, Wr1, b1, Wl, Wr, b):
    srcs, dsts = [], []
    for r, e in enumerate((e0, e1, e2, e3, e4, e5)):
        src = e[0].astype(jnp.int32) + r * NP
        dst = e[1].astype(jnp.int32)
        pad_src = jnp.full((E_PAD - E,), r * NP, jnp.int32)
        pad_dst = jnp.full((E_PAD - E,), N, jnp.int32)
        srcs.append(jnp.concatenate([src, pad_src]))
        dsts.append(jnp.concatenate([dst, pad_dst]))
    src_all = jnp.stack(srcs).reshape(6, NS * NCHK, CH)
    dst_all = jnp.stack(dsts).reshape(6, NS * NCHK, CH)

    zeros64 = jnp.zeros((SLICE, D_H), jnp.float32)
    zeros16 = jnp.zeros((SLICE, 16), jnp.float32)
    ones16 = jnp.ones((CH, 16), jnp.float32)

    cnt = _sc_count(dst_all, ones16, zeros16)

    pad_rows = ((0, NP - N), (0, 0))
    xc = jnp.pad(x_c, pad_rows)
    xm = jnp.pad(x_m, pad_rows)
    xd = jnp.pad(x_d, pad_rows)

    y, root = _tc_first(xc, xm, xd, Wl1, Wr1, b1)
    for i in range(3):
        s = _sc_aggregate(y.reshape(6 * NP, D_H), src_all, dst_all, zeros64)
        y, root = _tc_mid(s, cnt, root, Wl[i], Wr[i], b[i])
    s = _sc_aggregate(y.reshape(6 * NP, D_H), src_all, dst_all, zeros64)
    out = _tc_final(s, cnt, root)
    return out[:, :N, :]


# speedup vs baseline: 6.0337x; 1.0188x over previous
import functools

import jax
import jax.numpy as jnp
from jax import lax
from jax.experimental import pallas as pl
from jax.experimental.pallas import tpu as pltpu
from jax.experimental.pallas import tpu_sc as plsc

N = 10000
E = 160000
D_IN = 128
D_H = 64

NC = 2
NS = 16
NP = 10112
SLICE = NP // NS
CH = 128
EPT = (E + NS - 1) // NS
NCHK = -(-(-(-EPT // CH)) // 8) * 8
CHG = 2 * CH
NCHKG = NCHK // 2
EPT_PAD = NCHK * CH
E_PAD = EPT_PAD * NS

BN = 1264
GRID = NP // BN

SRC_T = (0, 1, 0, 1, 2, 2)
PAIRS = ((3, 5), (0, 4), (1, 2))

_mesh = plsc.VectorSubcoreMesh(
    core_axis_name="c", subcore_axis_name="s", num_cores=NC, num_subcores=NS)
_sc_params = pltpu.CompilerParams(use_tc_tiling_on_sc=False)


@functools.partial(
    pl.kernel,
    out_type=jax.ShapeDtypeStruct((6, NP, D_H), jnp.float32),
    mesh=_mesh,
    compiler_params=_sc_params,
    scratch_types=[
        pltpu.VMEM((NCHKG, CHG), jnp.int32),
        pltpu.VMEM((NCHK, CH), jnp.int32),
        [pltpu.VMEM((CHG, D_H), jnp.float32) for _ in range(4)],
        pltpu.VMEM_SHARED((NP, D_H), jnp.float32),
        [pltpu.SemaphoreType.DMA for _ in range(4)],
    ],
)
def _sc_aggregate(y_hbm, srcg_hbm, dst_hbm, z_hbm, out_hbm,
                  sidx, didx, rows, acc, sems):
    c = lax.axis_index("c")
    s = lax.axis_index("s")
    gsem_a, ssem_a, gsem_b, ssem_b = sems
    bufs_a, bufs_b = rows[:2], rows[2:]

    def start_gathers(g0, bufs, sem):
        for i in range(2):
            pltpu.async_copy(y_hbm.at[sidx.at[g0 + i]], bufs[i], sem)

    def drain_gathers(g0, bufs, sem):
        for i in range(2):
            pltpu.make_async_copy(y_hbm.at[sidx.at[g0 + i]], bufs[i],
                                  sem).wait()

    def run_scatters(g0, bufs, sem):
        descs = []
        for i in range(2):
            for h in range(2):
                descs.append(pltpu.async_copy(
                    bufs[i].at[pl.ds(h * CH, CH)],
                    acc.at[didx.at[2 * (g0 + i) + h]], sem, add=True))
        for d in descs:
            d.wait()

    for rl in range(3):
        r = c * 3 + rl
        pltpu.sync_copy(z_hbm, acc.at[pl.ds(s * SLICE, SLICE)])
        pltpu.sync_copy(srcg_hbm.at[r, pl.ds(s * NCHKG, NCHKG)], sidx)
        pltpu.sync_copy(dst_hbm.at[r, pl.ds(s * NCHK, NCHK)], didx)
        plsc.subcore_barrier()

        start_gathers(0, bufs_a, gsem_a)

        def step(j, carry):
            g = j * 4
            start_gathers(g + 2, bufs_b, gsem_b)
            drain_gathers(g, bufs_a, gsem_a)
            run_scatters(g, bufs_a, ssem_a)

            @pl.when(j < NCHKG // 4 - 1)
            def _():
                start_gathers(g + 4, bufs_a, gsem_a)

            drain_gathers(g + 2, bufs_b, gsem_b)
            run_scatters(g + 2, bufs_b, ssem_b)
            return carry

        lax.fori_loop(0, NCHKG // 4, step, 0)
        plsc.subcore_barrier()
        pltpu.sync_copy(
            acc.at[pl.ds(s * SLICE, SLICE)],
            out_hbm.at[r, pl.ds(s * SLICE, SLICE)])


@functools.partial(
    pl.kernel,
    out_type=jax.ShapeDtypeStruct((6, NP, 16), jnp.float32),
    mesh=_mesh,
    compiler_params=_sc_params,
    scratch_types=[
        pltpu.VMEM((NCHK, CH), jnp.int32),
        pltpu.VMEM((CH, 16), jnp.float32),
        pltpu.VMEM_SHARED((NP, 16), jnp.float32),
    ],
)
def _sc_count(dst_hbm, ones_hbm, z_hbm, out_hbm, didx, ones_v, acc):
    c = lax.axis_index("c")
    s = lax.axis_index("s")
    pltpu.sync_copy(ones_hbm, ones_v)
    for rl in range(3):
        r = c * 3 + rl
        pltpu.sync_copy(z_hbm, acc.at[pl.ds(s * SLICE, SLICE)])
        pltpu.sync_copy(dst_hbm.at[r, pl.ds(s * NCHK, NCHK)], didx)
        plsc.subcore_barrier()

        def chunk(k, carry):
            pltpu.sync_copy(ones_v, acc.at[didx.at[k]], add=True)
            return carry

        lax.fori_loop(0, NCHK, chunk, 0)
        plsc.subcore_barrier()
        pltpu.sync_copy(
            acc.at[pl.ds(s * SLICE, SLICE)],
            out_hbm.at[r, pl.ds(s * SLICE, SLICE)])


def _matT(x, w):
    return lax.dot_general(x, w, (((1,), (1,)), ((), ())),
                           preferred_element_type=jnp.float32)


def _tc_first_body(xc_ref, xm_ref, xd_ref, wl_ref, wr_ref, b_ref,
                   y_ref, r_ref):
    xs = [xc_ref[...], xm_ref[...], xd_ref[...]]
    bfull = b_ref[...]
    for r in range(6):
        y_ref[r] = _matT(xs[SRC_T[r]], wl_ref[r])
    for t, (a, b2) in enumerate(PAIRS):
        wrm = 0.5 * (wr_ref[a] + wr_ref[b2])
        bm = 0.5 * (bfull[a:a + 1, :] + bfull[b2:b2 + 1, :])
        r_ref[t] = _matT(xs[t], wrm) + bm


def _combine(s_ref, cnt_ref, root_ref, t, relu):
    a, b2 = PAIRS[t]
    inva = 1.0 / jnp.maximum(cnt_ref[a, :, 0:1], 1.0)
    invb = 1.0 / jnp.maximum(cnt_ref[b2, :, 0:1], 1.0)
    h = 0.5 * (s_ref[a] * inva + s_ref[b2] * invb) + root_ref[t]
    return jnp.maximum(h, 0.0) if relu else h


def _tc_mid_body(s_ref, cnt_ref, root_ref, wl_ref, wr_ref, b_ref,
                 y_ref, r_ref):
    hs = [_combine(s_ref, cnt_ref, root_ref, t, True) for t in range(3)]
    bfull = b_ref[...]
    for r in range(6):
        y_ref[r] = _matT(hs[SRC_T[r]], wl_ref[r])
    for t, (a, b2) in enumerate(PAIRS):
        wrm = 0.5 * (wr_ref[a] + wr_ref[b2])
        bm = 0.5 * (bfull[a:a + 1, :] + bfull[b2:b2 + 1, :])
        r_ref[t] = _matT(hs[t], wrm) + bm


def _tc_final_body(s_ref, cnt_ref, root_ref, out_ref):
    for t in range(3):
        h = _combine(s_ref, cnt_ref, root_ref, t, False)
        nrm = jnp.sqrt(jnp.sum(h * h, axis=1, keepdims=True))
        out_ref[t] = h / jnp.maximum(nrm, 1e-12)


def _full(shape):
    nd = len(shape)
    return pl.BlockSpec(shape, lambda i, _n=nd: (0,) * _n)


def _tc_first(xc, xm, xd, wl, wr, b):
    return pl.pallas_call(
        _tc_first_body,
        grid=(GRID,),
        in_specs=[
            pl.BlockSpec((BN, D_IN), lambda i: (i, 0)),
            pl.BlockSpec((BN, D_IN), lambda i: (i, 0)),
            pl.BlockSpec((BN, D_IN), lambda i: (i, 0)),
            _full(wl.shape), _full(wr.shape), _full(b.shape),
        ],
        out_specs=[
            pl.BlockSpec((6, BN, D_H), lambda i: (0, i, 0)),
            pl.BlockSpec((3, BN, D_H), lambda i: (0, i, 0)),
        ],
        out_shape=[
            jax.ShapeDtypeStruct((6, NP, D_H), jnp.float32),
            jax.ShapeDtypeStruct((3, NP, D_H), jnp.float32),
        ],
    )(xc, xm, xd, wl, wr, b)


def _tc_mid(s, cnt, root, wl, wr, b):
    return pl.pallas_call(
        _tc_mid_body,
        grid=(GRID,),
        in_specs=[
            pl.BlockSpec((6, BN, D_H), lambda i: (0, i, 0)),
            pl.BlockSpec((6, BN, 16), lambda i: (0, i, 0)),
            pl.BlockSpec((3, BN, D_H), lambda i: (0, i, 0)),
            _full(wl.shape), _full(wr.shape), _full(b.shape),
        ],
        out_specs=[
            pl.BlockSpec((6, BN, D_H), lambda i: (0, i, 0)),
            pl.BlockSpec((3, BN, D_H), lambda i: (0, i, 0)),
        ],
        out_shape=[
            jax.ShapeDtypeStruct((6, NP, D_H), jnp.float32),
            jax.ShapeDtypeStruct((3, NP, D_H), jnp.float32),
        ],
    )(s, cnt, root, wl, wr, b)


def _tc_final(s, cnt, root):
    return pl.pallas_call(
        _tc_final_body,
        grid=(GRID,),
        in_specs=[
            pl.BlockSpec((6, BN, D_H), lambda i: (0, i, 0)),
            pl.BlockSpec((6, BN, 16), lambda i: (0, i, 0)),
            pl.BlockSpec((3, BN, D_H), lambda i: (0, i, 0)),
        ],
        out_specs=pl.BlockSpec((3, BN, D_H), lambda i: (0, i, 0)),
        out_shape=jax.ShapeDtypeStruct((3, NP, D_H), jnp.float32),
    )(s, cnt, root)


def kernel(x_c, x_m, x_d, e0, e1, e2, e3, e4, e5, Wl1, Wr1, b1, Wl, Wr, b):
    srcs, dsts = [], []
    for r, e in enumerate((e0, e1, e2, e3, e4, e5)):
        src = e[0].astype(jnp.int32) + r * NP
        dst = e[1].astype(jnp.int32)
        pad_src = jnp.full((E_PAD - E,), r * NP, jnp.int32)
        pad_dst = jnp.full((E_PAD - E,), N, jnp.int32)
        srcs.append(jnp.concatenate([src, pad_src]))
        dsts.append(jnp.concatenate([dst, pad_dst]))
    src_all = jnp.stack(srcs).reshape(6, NS * NCHKG, CHG)
    dst_all = jnp.stack(dsts).reshape(6, NS * NCHK, CH)

    zeros64 = jnp.zeros((SLICE, D_H), jnp.float32)
    zeros16 = jnp.zeros((SLICE, 16), jnp.float32)
    ones16 = jnp.ones((CH, 16), jnp.float32)

    cnt = _sc_count(dst_all, ones16, zeros16)

    pad_rows = ((0, NP - N), (0, 0))
    xc = jnp.pad(x_c, pad_rows)
    xm = jnp.pad(x_m, pad_rows)
    xd = jnp.pad(x_d, pad_rows)

    y, root = _tc_first(xc, xm, xd, Wl1, Wr1, b1)
    for i in range(3):
        s = _sc_aggregate(y.reshape(6 * NP, D_H), src_all, dst_all, zeros64)
        y, root = _tc_mid(s, cnt, root, Wl[i], Wr[i], b[i])
    s = _sc_aggregate(y.reshape(6 * NP, D_H), src_all, dst_all, zeros64)
    out = _tc_final(s, cnt, root)
    return out[:, :N, :]
